# Initial kernel scaffold; baseline (speedup 1.0000x reference)
import jax
import jax.numpy as jnp
from jax.experimental import pallas as pl


def kernel(x, topological_features, edge_index, batch_size, W_dgi, alpha, W_down):
    raise NotImplementedError("write your pallas kernel here")


# speedup vs baseline: 3.9640x; 3.9640x over previous
import functools

import jax
import jax.numpy as jnp
from jax import lax
from jax.experimental import pallas as pl
from jax.experimental.pallas import tpu as pltpu
from jax.experimental.pallas import tpu_sc as plsc

N = 10000
E = 320000
D_FEAT = 128
D_TOPO = 64
D_LAT = 128
D_OUT = 128

NC = 2
NS = 16
NW = NC * NS

N_PAD = 10240
STRIPE = N_PAD // NS
WB = 128
WINS = 80
EPT = WB * WINS
E_PAD = NW * EPT
R_BLK = 1024


def _seg_body_p1(srcg, dstg, topo, z64, zd, outp, degp,
                 sidx, didx, rbuf, obuf, acc, dacc, gsem):
    c = lax.axis_index("c")
    s = lax.axis_index("s")
    wid = c * NS + s
    r0 = s * STRIPE
    pltpu.sync_copy(srcg.at[wid], sidx)
    pltpu.sync_copy(dstg.at[wid], didx)
    for i in range(8):
        obuf[pl.ds(i * 16, 16)] = jnp.ones((16,), jnp.float32)
    pltpu.sync_copy(z64.at[pl.ds(r0, STRIPE), :], acc.at[pl.ds(r0, STRIPE), :])
    pltpu.sync_copy(zd.at[pl.ds(r0, STRIPE)], dacc.at[pl.ds(r0, STRIPE)])
    plsc.subcore_barrier()

    def w_body(w, carry):
        pltpu.async_copy(topo.at[sidx.at[w]], rbuf, gsem).wait()
        pltpu.sync_copy(rbuf, acc.at[didx.at[w]], add=True)
        pltpu.sync_copy(obuf, dacc.at[didx.at[w]], add=True)
        return carry

    lax.fori_loop(0, WINS, w_body, 0)
    plsc.subcore_barrier()
    out_r0 = c * N_PAD + r0
    pltpu.sync_copy(acc.at[pl.ds(r0, STRIPE), :], outp.at[pl.ds(out_r0, STRIPE), :])
    pltpu.sync_copy(dacc.at[pl.ds(r0, STRIPE)], degp.at[pl.ds(out_r0, STRIPE)])


def _seg_body_p2(srcg, dstg, table, z128, outp,
                 sidx, didx, rbuf, acc, gsem):
    c = lax.axis_index("c")
    s = lax.axis_index("s")
    wid = c * NS + s
    r0 = s * STRIPE
    pltpu.sync_copy(srcg.at[wid], sidx)
    pltpu.sync_copy(dstg.at[wid], didx)
    pltpu.sync_copy(z128.at[pl.ds(r0, STRIPE), :], acc.at[pl.ds(r0, STRIPE), :])
    plsc.subcore_barrier()

    def w_body(w, carry):
        pltpu.async_copy(table.at[sidx.at[w]], rbuf, gsem).wait()
        pltpu.sync_copy(rbuf, acc.at[didx.at[w]], add=True)
        return carry

    lax.fori_loop(0, WINS, w_body, 0)
    plsc.subcore_barrier()
    out_r0 = c * N_PAD + r0
    pltpu.sync_copy(acc.at[pl.ds(r0, STRIPE), :], outp.at[pl.ds(out_r0, STRIPE), :])


_MESH = plsc.VectorSubcoreMesh(core_axis_name="c", subcore_axis_name="s")
_SC_PARAMS = pltpu.CompilerParams(use_tc_tiling_on_sc=False)

_seg_p1 = pl.kernel(
    _seg_body_p1,
    out_type=(
        jax.ShapeDtypeStruct((NC * N_PAD, D_TOPO), jnp.float32),
        jax.ShapeDtypeStruct((NC * N_PAD,), jnp.float32),
    ),
    mesh=_MESH,
    scratch_types=[
        pltpu.VMEM((WINS, WB), jnp.int32),
        pltpu.VMEM((WINS, WB), jnp.int32),
        pltpu.VMEM((WB, D_TOPO), jnp.float32),
        pltpu.VMEM((WB,), jnp.float32),
        pltpu.VMEM_SHARED((N_PAD, D_TOPO), jnp.float32),
        pltpu.VMEM_SHARED((N_PAD,), jnp.float32),
        pltpu.SemaphoreType.DMA,
    ],
    compiler_params=_SC_PARAMS,
)

_seg_p2 = pl.kernel(
    _seg_body_p2,
    out_type=jax.ShapeDtypeStruct((NC * N_PAD, D_LAT), jnp.float32),
    mesh=_MESH,
    scratch_types=[
        pltpu.VMEM((WINS, WB), jnp.int32),
        pltpu.VMEM((WINS, WB), jnp.int32),
        pltpu.VMEM((WB, D_LAT), jnp.float32),
        pltpu.VMEM_SHARED((N_PAD, D_LAT), jnp.float32),
        pltpu.SemaphoreType.DMA,
    ],
    compiler_params=_SC_PARAMS,
)


def _dense_a_body(p_ref, dg_ref, x_ref, wd_ref, wdn_ref, a_ref, h2_ref, inv_ref):
    a = p_ref[0] + p_ref[1]
    draw = dg_ref[0] + dg_ref[1]
    deg = jnp.maximum(draw, 1.0)
    h = jnp.dot(a, wd_ref[...], preferred_element_type=jnp.float32) / deg
    alpha = a_ref[0, 0]
    lat = jnp.where(h > 0, h, alpha * h)
    w1 = wdn_ref[0:D_FEAT, :]
    w2 = wdn_ref[D_FEAT:, :]
    h2 = (jnp.dot(x_ref[...], w1, preferred_element_type=jnp.float32)
          + jnp.dot(lat, w2, preferred_element_type=jnp.float32))
    h2_ref[...] = h2
    inv_ref[...] = jnp.broadcast_to(1.0 / (deg + 1.0), h2.shape)


def _dense_b_body(p_ref, h2_ref, inv_ref, o_ref):
    o_ref[...] = (p_ref[0] + p_ref[1] + h2_ref[...]) * inv_ref[...]


@jax.jit
def _impl(x, topological_features, edge_index, W_dgi, alpha, W_down):
    f32 = jnp.float32
    src = edge_index[0]
    dst = edge_index[1]
    pe = E_PAD - E
    src_p = jnp.concatenate([src, jnp.zeros((pe,), jnp.int32)])
    dst_p = jnp.concatenate(
        [dst, N + (jnp.arange(pe, dtype=jnp.int32) % (N_PAD - N))])
    srcg = src_p.reshape(NW, WINS, WB)
    dstg = dst_p.reshape(NW, WINS, WB)

    z64 = jnp.zeros((N_PAD, D_TOPO), f32)
    zd = jnp.zeros((N_PAD,), f32)
    z128 = jnp.zeros((N_PAD, D_LAT), f32)

    p1, degp = _seg_p1(srcg, dstg, topological_features, z64, zd)

    x_pad = jnp.pad(x, ((0, N_PAD - N), (0, 0)))
    nblk = N_PAD // R_BLK
    h2, invb = pl.pallas_call(
        _dense_a_body,
        grid=(nblk,),
        in_specs=[
            pl.BlockSpec((2, R_BLK, D_TOPO), lambda i: (0, i, 0)),
            pl.BlockSpec((2, R_BLK, 1), lambda i: (0, i, 0)),
            pl.BlockSpec((R_BLK, D_FEAT), lambda i: (i, 0)),
            pl.BlockSpec((D_TOPO, D_LAT), lambda i: (0, 0)),
            pl.BlockSpec((D_FEAT + D_LAT, D_OUT), lambda i: (0, 0)),
            pl.BlockSpec((1, 1), lambda i: (0, 0)),
        ],
        out_specs=[
            pl.BlockSpec((R_BLK, D_OUT), lambda i: (i, 0)),
            pl.BlockSpec((R_BLK, D_OUT), lambda i: (i, 0)),
        ],
        out_shape=[
            jax.ShapeDtypeStruct((N_PAD, D_OUT), f32),
            jax.ShapeDtypeStruct((N_PAD, D_OUT), f32),
        ],
    )(p1.reshape(NC, N_PAD, D_TOPO), degp.reshape(NC, N_PAD, 1), x_pad,
      W_dgi, W_down, alpha.reshape(1, 1))

    p2 = _seg_p2(srcg, dstg, h2, z128)

    out = pl.pallas_call(
        _dense_b_body,
        grid=(nblk,),
        in_specs=[
            pl.BlockSpec((2, R_BLK, D_OUT), lambda i: (0, i, 0)),
            pl.BlockSpec((R_BLK, D_OUT), lambda i: (i, 0)),
            pl.BlockSpec((R_BLK, D_OUT), lambda i: (i, 0)),
        ],
        out_specs=pl.BlockSpec((R_BLK, D_OUT), lambda i: (i, 0)),
        out_shape=jax.ShapeDtypeStruct((N_PAD, D_OUT), f32),
    )(p2.reshape(NC, N_PAD, D_OUT), h2, invb)

    return out[:N]


def kernel(x, topological_features, edge_index, batch_size, W_dgi, alpha, W_down):
    del batch_size
    return _impl(x, topological_features, edge_index, W_dgi, alpha, W_down)


# speedup vs baseline: 4.5951x; 1.1592x over previous
import functools

import jax
import jax.numpy as jnp
from jax import lax
from jax.experimental import pallas as pl
from jax.experimental.pallas import tpu as pltpu
from jax.experimental.pallas import tpu_sc as plsc

N = 10000
E = 320000
D_FEAT = 128
D_TOPO = 64
D_LAT = 128
D_OUT = 128

NC = 2
NS = 16
NW = NC * NS

N_PAD = 10240
STRIPE = N_PAD // NS
WB1 = 128
WINS1 = 80
WB2 = 64
WINS2 = 160
EPT = WB1 * WINS1
E_PAD = NW * EPT
R_BLK = 1024


def _seg_body_p1(WINS, srcg, dstg, topo, z64, zd, outp, degp,
                 sidx, didx, rbuf0, rbuf1, obuf, acc, dacc, gsem0, gsem1):
    c = lax.axis_index("c")
    s = lax.axis_index("s")
    wid = c * NS + s
    r0 = s * STRIPE
    pltpu.sync_copy(srcg.at[wid], sidx)
    pltpu.sync_copy(dstg.at[wid], didx)
    for i in range(8):
        obuf[pl.ds(i * 16, 16)] = jnp.ones((16,), jnp.float32)
    pltpu.sync_copy(z64.at[pl.ds(r0, STRIPE), :], acc.at[pl.ds(r0, STRIPE), :])
    pltpu.sync_copy(zd.at[pl.ds(r0, STRIPE)], dacc.at[pl.ds(r0, STRIPE)])
    plsc.subcore_barrier()

    pltpu.async_copy(topo.at[sidx.at[0]], rbuf0, gsem0)
    pltpu.async_copy(topo.at[sidx.at[1]], rbuf1, gsem1)

    def pair_body(p, carry):
        w = 2 * p
        pltpu.make_async_copy(topo.at[sidx.at[w]], rbuf0, gsem0).wait()
        pltpu.sync_copy(rbuf0, acc.at[didx.at[w]], add=True)
        pltpu.sync_copy(obuf, dacc.at[didx.at[w]], add=True)
        pltpu.async_copy(topo.at[sidx.at[w + 2]], rbuf0, gsem0)
        pltpu.make_async_copy(topo.at[sidx.at[w + 1]], rbuf1, gsem1).wait()
        pltpu.sync_copy(rbuf1, acc.at[didx.at[w + 1]], add=True)
        pltpu.sync_copy(obuf, dacc.at[didx.at[w + 1]], add=True)
        pltpu.async_copy(topo.at[sidx.at[w + 3]], rbuf1, gsem1)
        return carry

    lax.fori_loop(0, WINS // 2 - 1, pair_body, 0)
    pltpu.make_async_copy(topo.at[sidx.at[0]], rbuf0, gsem0).wait()
    pltpu.sync_copy(rbuf0, acc.at[didx.at[WINS - 2]], add=True)
    pltpu.sync_copy(obuf, dacc.at[didx.at[WINS - 2]], add=True)
    pltpu.make_async_copy(topo.at[sidx.at[0]], rbuf1, gsem1).wait()
    pltpu.sync_copy(rbuf1, acc.at[didx.at[WINS - 1]], add=True)
    pltpu.sync_copy(obuf, dacc.at[didx.at[WINS - 1]], add=True)
    plsc.subcore_barrier()
    out_r0 = c * N_PAD + r0
    pltpu.sync_copy(acc.at[pl.ds(r0, STRIPE), :], outp.at[pl.ds(out_r0, STRIPE), :])
    pltpu.sync_copy(dacc.at[pl.ds(r0, STRIPE)], degp.at[pl.ds(out_r0, STRIPE)])


def _seg_body_p2(WINS, srcg, dstg, table, z128, outp,
                 sidx, didx, rbuf0, rbuf1, acc, gsem0, gsem1):
    c = lax.axis_index("c")
    s = lax.axis_index("s")
    wid = c * NS + s
    r0 = s * STRIPE
    pltpu.sync_copy(srcg.at[wid], sidx)
    pltpu.sync_copy(dstg.at[wid], didx)
    pltpu.sync_copy(z128.at[pl.ds(r0, STRIPE), :], acc.at[pl.ds(r0, STRIPE), :])
    plsc.subcore_barrier()

    pltpu.async_copy(table.at[sidx.at[0]], rbuf0, gsem0)
    pltpu.async_copy(table.at[sidx.at[1]], rbuf1, gsem1)

    def pair_body(p, carry):
        w = 2 * p
        pltpu.make_async_copy(table.at[sidx.at[w]], rbuf0, gsem0).wait()
        pltpu.sync_copy(rbuf0, acc.at[didx.at[w]], add=True)
        pltpu.async_copy(table.at[sidx.at[w + 2]], rbuf0, gsem0)
        pltpu.make_async_copy(table.at[sidx.at[w + 1]], rbuf1, gsem1).wait()
        pltpu.sync_copy(rbuf1, acc.at[didx.at[w + 1]], add=True)
        pltpu.async_copy(table.at[sidx.at[w + 3]], rbuf1, gsem1)
        return carry

    lax.fori_loop(0, WINS // 2 - 1, pair_body, 0)
    pltpu.make_async_copy(table.at[sidx.at[0]], rbuf0, gsem0).wait()
    pltpu.sync_copy(rbuf0, acc.at[didx.at[WINS - 2]], add=True)
    pltpu.make_async_copy(table.at[sidx.at[0]], rbuf1, gsem1).wait()
    pltpu.sync_copy(rbuf1, acc.at[didx.at[WINS - 1]], add=True)
    plsc.subcore_barrier()
    out_r0 = c * N_PAD + r0
    pltpu.sync_copy(acc.at[pl.ds(r0, STRIPE), :], outp.at[pl.ds(out_r0, STRIPE), :])


_MESH = plsc.VectorSubcoreMesh(core_axis_name="c", subcore_axis_name="s")
_SC_PARAMS = pltpu.CompilerParams(use_tc_tiling_on_sc=False)

_seg_p1 = pl.kernel(
    functools.partial(_seg_body_p1, WINS1),
    out_type=(
        jax.ShapeDtypeStruct((NC * N_PAD, D_TOPO), jnp.float32),
        jax.ShapeDtypeStruct((NC * N_PAD,), jnp.float32),
    ),
    mesh=_MESH,
    scratch_types=[
        pltpu.VMEM((WINS1, WB1), jnp.int32),
        pltpu.VMEM((WINS1, WB1), jnp.int32),
        pltpu.VMEM((WB1, D_TOPO), jnp.float32),
        pltpu.VMEM((WB1, D_TOPO), jnp.float32),
        pltpu.VMEM((WB1,), jnp.float32),
        pltpu.VMEM_SHARED((N_PAD, D_TOPO), jnp.float32),
        pltpu.VMEM_SHARED((N_PAD,), jnp.float32),
        pltpu.SemaphoreType.DMA,
        pltpu.SemaphoreType.DMA,
    ],
    compiler_params=_SC_PARAMS,
)

_seg_p2 = pl.kernel(
    functools.partial(_seg_body_p2, WINS2),
    out_type=jax.ShapeDtypeStruct((NC * N_PAD, D_LAT), jnp.float32),
    mesh=_MESH,
    scratch_types=[
        pltpu.VMEM((WINS2, WB2), jnp.int32),
        pltpu.VMEM((WINS2, WB2), jnp.int32),
        pltpu.VMEM((WB2, D_LAT), jnp.float32),
        pltpu.VMEM((WB2, D_LAT), jnp.float32),
        pltpu.VMEM_SHARED((N_PAD, D_LAT), jnp.float32),
        pltpu.SemaphoreType.DMA,
        pltpu.SemaphoreType.DMA,
    ],
    compiler_params=_SC_PARAMS,
)


def _dense_a_body(p_ref, dg_ref, x_ref, wd_ref, wdn_ref, a_ref, h2_ref, inv_ref):
    a = p_ref[0] + p_ref[1]
    draw = dg_ref[0] + dg_ref[1]
    deg = jnp.maximum(draw, 1.0)
    h = jnp.dot(a, wd_ref[...], preferred_element_type=jnp.float32) / deg
    alpha = a_ref[0, 0]
    lat = jnp.where(h > 0, h, alpha * h)
    w1 = wdn_ref[0:D_FEAT, :]
    w2 = wdn_ref[D_FEAT:, :]
    h2 = (jnp.dot(x_ref[...], w1, preferred_element_type=jnp.float32)
          + jnp.dot(lat, w2, preferred_element_type=jnp.float32))
    h2_ref[...] = h2
    inv_ref[...] = jnp.broadcast_to(1.0 / (deg + 1.0), h2.shape)


def _dense_b_body(p_ref, h2_ref, inv_ref, o_ref):
    o_ref[...] = (p_ref[0] + p_ref[1] + h2_ref[...]) * inv_ref[...]


@jax.jit
def _impl(x, topological_features, edge_index, W_dgi, alpha, W_down):
    f32 = jnp.float32
    src = edge_index[0]
    dst = edge_index[1]
    pe = E_PAD - E
    src_p = jnp.concatenate([src, jnp.zeros((pe,), jnp.int32)])
    dst_p = jnp.concatenate(
        [dst, N + (jnp.arange(pe, dtype=jnp.int32) % (N_PAD - N))])
    srcg1 = src_p.reshape(NW, WINS1, WB1)
    dstg1 = dst_p.reshape(NW, WINS1, WB1)
    srcg2 = src_p.reshape(NW, WINS2, WB2)
    dstg2 = dst_p.reshape(NW, WINS2, WB2)

    z64 = jnp.zeros((N_PAD, D_TOPO), f32)
    zd = jnp.zeros((N_PAD,), f32)
    z128 = jnp.zeros((N_PAD, D_LAT), f32)

    p1, degp = _seg_p1(srcg1, dstg1, topological_features, z64, zd)

    x_pad = jnp.pad(x, ((0, N_PAD - N), (0, 0)))
    nblk = N_PAD // R_BLK
    h2, invb = pl.pallas_call(
        _dense_a_body,
        grid=(nblk,),
        in_specs=[
            pl.BlockSpec((2, R_BLK, D_TOPO), lambda i: (0, i, 0)),
            pl.BlockSpec((2, R_BLK, 1), lambda i: (0, i, 0)),
            pl.BlockSpec((R_BLK, D_FEAT), lambda i: (i, 0)),
            pl.BlockSpec((D_TOPO, D_LAT), lambda i: (0, 0)),
            pl.BlockSpec((D_FEAT + D_LAT, D_OUT), lambda i: (0, 0)),
            pl.BlockSpec((1, 1), lambda i: (0, 0)),
        ],
        out_specs=[
            pl.BlockSpec((R_BLK, D_OUT), lambda i: (i, 0)),
            pl.BlockSpec((R_BLK, D_OUT), lambda i: (i, 0)),
        ],
        out_shape=[
            jax.ShapeDtypeStruct((N_PAD, D_OUT), f32),
            jax.ShapeDtypeStruct((N_PAD, D_OUT), f32),
        ],
    )(p1.reshape(NC, N_PAD, D_TOPO), degp.reshape(NC, N_PAD, 1), x_pad,
      W_dgi, W_down, alpha.reshape(1, 1))

    p2 = _seg_p2(srcg2, dstg2, h2, z128)

    out = pl.pallas_call(
        _dense_b_body,
        grid=(nblk,),
        in_specs=[
            pl.BlockSpec((2, R_BLK, D_OUT), lambda i: (0, i, 0)),
            pl.BlockSpec((R_BLK, D_OUT), lambda i: (i, 0)),
            pl.BlockSpec((R_BLK, D_OUT), lambda i: (i, 0)),
        ],
        out_specs=pl.BlockSpec((R_BLK, D_OUT), lambda i: (i, 0)),
        out_shape=jax.ShapeDtypeStruct((N_PAD, D_OUT), f32),
    )(p2.reshape(NC, N_PAD, D_OUT), h2, invb)

    return out[:N]


def kernel(x, topological_features, edge_index, batch_size, W_dgi, alpha, W_down):
    del batch_size
    return _impl(x, topological_features, edge_index, W_dgi, alpha, W_down)
